# baseline (speedup 1.0000x reference)
import functools

import jax
import jax.numpy as jnp
from jax import lax
from jax.experimental import pallas as pl
from jax.experimental.pallas import tpu as pltpu
from jax.experimental.pallas import tpu_sc as plsc

N_MID = 1000000
DIM = 64
SEQ = 50
BATCH = 4096
ROWS = BATCH * SEQ

NC = 2
NS = 16
NW = NC * NS
LANES = 16

PDIM = 128

IDX_MINOR = 128
IDX_ROWS = ROWS // IDX_MINOR
IDXR_PER_W = IDX_ROWS // NW
IDXR_PER_CHUNK = 2
N_CHUNKS = IDXR_PER_W // IDXR_PER_CHUNK
CHUNK = IDXR_PER_CHUNK * IDX_MINOR

TC_BLOCK_SEQS = 64


def _sc_gather(idx_hbm, table_hbm, out_hbm, idx_v, rows_v, gsem, ssem0, ssem1):
    wid = lax.axis_index("s") * NC + lax.axis_index("c")
    idxr0 = wid * IDXR_PER_W
    row0 = wid * IDXR_PER_W * IDX_MINOR

    ssems = (ssem0, ssem1)
    store_handles = [None, None]
    for k in range(N_CHUNKS):
        p = k % 2
        if store_handles[p] is not None:
            store_handles[p].wait()
        pltpu.sync_copy(
            idx_hbm.at[pl.ds(idxr0 + k * IDXR_PER_CHUNK, IDXR_PER_CHUNK)],
            idx_v.at[p],
        )
        gathers = []
        for j in range(IDXR_PER_CHUNK):
            gathers.append(
                pltpu.async_copy(
                    table_hbm.at[idx_v.at[p, j]],
                    rows_v.at[p, pl.ds(j * IDX_MINOR, IDX_MINOR)],
                    gsem,
                )
            )
        for g in gathers:
            g.wait()
        store_handles[p] = pltpu.async_copy(
            rows_v.at[p],
            out_hbm.at[pl.ds(row0 + k * CHUNK, CHUNK)],
            ssems[p],
        )
    for h in store_handles:
        if h is not None:
            h.wait()


def _tc_add(rows_ref, pos_ref, out_ref):
    for b in range(TC_BLOCK_SEQS):
        out_ref[b] = (
            rows_ref[pl.ds(b * SEQ, SEQ), pl.ds(0, DIM)]
            + pos_ref[:, pl.ds(0, DIM)]
        )


def kernel(item, nbr_mask, i_ids, item_input_lookup, position_embedding):
    del nbr_mask, i_ids

    mesh = plsc.VectorSubcoreMesh(core_axis_name="c", subcore_axis_name="s")

    dense2d = jnp.concatenate([item_input_lookup, item_input_lookup], axis=1)

    idx2d = item.reshape(IDX_ROWS, IDX_MINOR)
    gather = functools.partial(
        pl.kernel,
        mesh=mesh,
        out_type=jax.ShapeDtypeStruct((ROWS, PDIM), jnp.float32),
        scratch_types=[
            pltpu.VMEM((2, IDXR_PER_CHUNK, IDX_MINOR), jnp.int32),
            pltpu.VMEM((2, CHUNK, PDIM), jnp.float32),
            pltpu.SemaphoreType.DMA,
            pltpu.SemaphoreType.DMA,
            pltpu.SemaphoreType.DMA,
        ],
        compiler_params=pltpu.CompilerParams(use_tc_tiling_on_sc=False),
    )(_sc_gather)
    gathered = gather(idx2d, dense2d)

    pos_pad = jnp.pad(position_embedding.reshape(SEQ, DIM), ((0, 0), (0, PDIM - DIM)))
    out = pl.pallas_call(
        _tc_add,
        grid=(BATCH // TC_BLOCK_SEQS,),
        in_specs=[
            pl.BlockSpec((TC_BLOCK_SEQS * SEQ, PDIM), lambda i: (i, 0)),
            pl.BlockSpec((SEQ, PDIM), lambda i: (0, 0)),
        ],
        out_specs=pl.BlockSpec((TC_BLOCK_SEQS, SEQ, DIM), lambda i: (i, 0, 0)),
        out_shape=jax.ShapeDtypeStruct((BATCH, SEQ, DIM), jnp.float32),
    )(gathered, pos_pad)
    return out


# speedup vs baseline: 1.1892x; 1.1892x over previous
import functools

import jax
import jax.numpy as jnp
from jax import lax
from jax.experimental import pallas as pl
from jax.experimental.pallas import tpu as pltpu
from jax.experimental.pallas import tpu_sc as plsc

N_MID = 1000000
DIM = 64
SEQ = 50
BATCH = 4096
ROWS = BATCH * SEQ

NC = 2
NS = 16
NW = NC * NS
LANES = 16

PDIM = 128

IDX_MINOR = 128
IDX_ROWS = ROWS // IDX_MINOR
IDXR_PER_W = IDX_ROWS // NW
IDXR_PER_CHUNK = 2
N_CHUNKS = IDXR_PER_W // IDXR_PER_CHUNK
CHUNK = IDXR_PER_CHUNK * IDX_MINOR

TC_BLOCK_SEQS = 64


def _sc_gather(idx_hbm, table_hbm, out_hbm, idx_v, rows_v, gsem, ssem0, ssem1):
    wid = lax.axis_index("s") * NC + lax.axis_index("c")
    idxr0 = wid * IDXR_PER_W
    row0 = wid * IDXR_PER_W * IDX_MINOR

    ssems = (ssem0, ssem1)
    store_handles = [None, None]
    for k in range(N_CHUNKS):
        p = k % 2
        if store_handles[p] is not None:
            store_handles[p].wait()
        pltpu.sync_copy(
            idx_hbm.at[pl.ds(idxr0 + k * IDXR_PER_CHUNK, IDXR_PER_CHUNK)],
            idx_v.at[p],
        )
        gathers = []
        for j in range(IDXR_PER_CHUNK):
            gathers.append(
                pltpu.async_copy(
                    table_hbm.at[idx_v.at[p, j]],
                    rows_v.at[p, pl.ds(j * IDX_MINOR, IDX_MINOR)],
                    gsem,
                )
            )
        for g in gathers:
            g.wait()
        store_handles[p] = pltpu.async_copy(
            rows_v.at[p],
            out_hbm.at[pl.ds(row0 + k * CHUNK, CHUNK)],
            ssems[p],
        )
    for h in store_handles:
        if h is not None:
            h.wait()


def _tc_add(rows_ref, pos_ref, out_ref):
    for b in range(TC_BLOCK_SEQS):
        out_ref[b] = (
            rows_ref[pl.ds(b * SEQ, SEQ), pl.ds(0, DIM)]
            + pos_ref[:, pl.ds(0, DIM)]
        )


def kernel(item, nbr_mask, i_ids, item_input_lookup, position_embedding):
    del nbr_mask, i_ids

    mesh = plsc.VectorSubcoreMesh(core_axis_name="c", subcore_axis_name="s")

    dense2d = jnp.pad(item_input_lookup, ((0, 0), (0, PDIM - DIM)))

    idx2d = item.reshape(IDX_ROWS, IDX_MINOR)
    gather = functools.partial(
        pl.kernel,
        mesh=mesh,
        out_type=jax.ShapeDtypeStruct((ROWS, PDIM), jnp.float32),
        scratch_types=[
            pltpu.VMEM((2, IDXR_PER_CHUNK, IDX_MINOR), jnp.int32),
            pltpu.VMEM((2, CHUNK, PDIM), jnp.float32),
            pltpu.SemaphoreType.DMA,
            pltpu.SemaphoreType.DMA,
            pltpu.SemaphoreType.DMA,
        ],
        compiler_params=pltpu.CompilerParams(use_tc_tiling_on_sc=False),
    )(_sc_gather)
    gathered = gather(idx2d, dense2d)

    pos_pad = jnp.pad(position_embedding.reshape(SEQ, DIM), ((0, 0), (0, PDIM - DIM)))
    out = pl.pallas_call(
        _tc_add,
        grid=(BATCH // TC_BLOCK_SEQS,),
        in_specs=[
            pl.BlockSpec((TC_BLOCK_SEQS * SEQ, PDIM), lambda i: (i, 0)),
            pl.BlockSpec((SEQ, PDIM), lambda i: (0, 0)),
        ],
        out_specs=pl.BlockSpec((TC_BLOCK_SEQS, SEQ, DIM), lambda i: (i, 0, 0)),
        out_shape=jax.ShapeDtypeStruct((BATCH, SEQ, DIM), jnp.float32),
    )(gathered, pos_pad)
    return out
